# Initial kernel scaffold; baseline (speedup 1.0000x reference)
import jax
import jax.numpy as jnp
from jax.experimental import pallas as pl


def kernel(t):
    raise NotImplementedError("write your pallas kernel here")


# speedup vs baseline: 2.1421x; 2.1421x over previous
import functools

import jax
import jax.numpy as jnp
from jax import lax
from jax.experimental import pallas as pl
from jax.experimental.pallas import tpu as pltpu
from jax.experimental.pallas import tpu_sc as plsc

_TIMESTEPS = 300
_DIM = 128
_B = 16384

_INFO = plsc.get_sparse_core_info()
_NC, _NS = _INFO.num_cores, _INFO.num_subcores
_NW = _NC * _NS
_B_PER_W = _B // _NW
_CHUNK = 128
_NCHUNK = _B_PER_W // _CHUNK


def _build_table() -> jnp.ndarray:
    half = _DIM // 2
    b = (jnp.arange(_TIMESTEPS + 1, dtype=jnp.float32) / 10000.0)[:, None]
    e = (jnp.arange(half, dtype=jnp.float32) / _DIM)[None, :]
    emb = b ** e
    emb = jnp.stack((jnp.sin(emb), jnp.cos(emb)), axis=-1)
    return emb.reshape(_TIMESTEPS + 1, _DIM)


_MESH = plsc.VectorSubcoreMesh(core_axis_name="c", subcore_axis_name="s")


@functools.partial(
    pl.kernel,
    out_type=jax.ShapeDtypeStruct((_B, _DIM), jnp.float32),
    mesh=_MESH,
    scratch_types=[
        pltpu.VMEM((_B_PER_W,), jnp.int32),
        pltpu.VMEM((_B_PER_W, _DIM), jnp.float32),
        pltpu.SemaphoreType.DMA,
        pltpu.SemaphoreType.DMA,
    ],
)
def _gather_kernel(table_hbm, t_hbm, out_hbm, idx_v, rows_v, gsem, ssem):
    wid = lax.axis_index("s") * _NC + lax.axis_index("c")
    base = wid * _B_PER_W
    pltpu.sync_copy(t_hbm.at[pl.ds(base, _B_PER_W)], idx_v)
    for j in range(_NCHUNK):
        pltpu.async_copy(
            table_hbm.at[idx_v.at[pl.ds(j * _CHUNK, _CHUNK)]],
            rows_v.at[pl.ds(j * _CHUNK, _CHUNK)],
            gsem,
        )
    for j in range(_NCHUNK):
        pltpu.make_async_copy(
            table_hbm.at[idx_v.at[pl.ds(j * _CHUNK, _CHUNK)]],
            rows_v.at[pl.ds(j * _CHUNK, _CHUNK)],
            gsem,
        ).wait()
    pltpu.async_copy(rows_v, out_hbm.at[pl.ds(base, _B_PER_W)], ssem).wait()


@jax.jit
def kernel(t):
    table = _build_table()
    return _gather_kernel(table, t.astype(jnp.int32))
